# baseline (speedup 1.0000x reference)
import functools

import jax
import jax.numpy as jnp
from jax import lax
from jax.experimental import pallas as pl
from jax.experimental.pallas import tpu as pltpu
from jax.experimental.pallas import tpu_sc as plsc

_VOCAB = 100000
_DIM = 300
_BATCH = 4096
_MAXLEN = 50
_B = _BATCH * _MAXLEN
_W = 128


def _mask_and_midx(idx32, len2d):
    def body(idx_ref, len_ref, mask_ref, midx_ref):
        pos = lax.broadcasted_iota(jnp.int32, (_BATCH, _MAXLEN), 1)
        valid = pos < len_ref[...]
        mask_ref[...] = valid.astype(jnp.float32)
        midx_ref[...] = jnp.where(valid, idx_ref[...], _VOCAB)

    return pl.pallas_call(
        body,
        out_shape=(
            jax.ShapeDtypeStruct((_BATCH, _MAXLEN), jnp.float32),
            jax.ShapeDtypeStruct((_BATCH, _MAXLEN), jnp.int32),
        ),
    )(idx32, len2d)


_DPAD = 304
_NSC = 32
_BPW = _B // _NSC
_NWIN = _BPW // _W


def _sc_gather(table_ext, midx):
    mesh = plsc.VectorSubcoreMesh(core_axis_name="c", subcore_axis_name="s")

    @functools.partial(
        pl.kernel,
        out_type=jax.ShapeDtypeStruct((_B, _DPAD), jnp.float32),
        mesh=mesh,
        scratch_types=[
            pltpu.VMEM((_BPW,), jnp.int32),
            pltpu.VMEM((_W, _DPAD), jnp.float32),
            pltpu.VMEM((_W, _DPAD), jnp.float32),
            pltpu.SemaphoreType.DMA,
            pltpu.SemaphoreType.DMA,
            pltpu.SemaphoreType.DMA,
            pltpu.SemaphoreType.DMA,
        ],
        compiler_params=pltpu.CompilerParams(use_tc_tiling_on_sc=False),
    )
    def k(table_hbm, idx_hbm, out_hbm, idx_v, b0, b1, gs0, gs1, ws0, ws1):
        wid = lax.axis_index("s") * 2 + lax.axis_index("c")
        base = wid * _BPW
        pltpu.sync_copy(idx_hbm.at[pl.ds(base, _BPW)], idx_v)

        def gather_start(w, buf, sem):
            pltpu.async_copy(table_hbm.at[idx_v.at[pl.ds(w * _W, _W)]], buf, sem)

        def gather_wait(buf, sem):
            pltpu.make_async_copy(
                table_hbm.at[idx_v.at[pl.ds(0, _W)]], buf, sem
            ).wait()

        def wb_start(w, buf, sem):
            pltpu.async_copy(
                buf,
                out_hbm.at[pl.ds(base + w * _W, _W)],
                sem,
            )

        def wb_wait(buf, sem):
            pltpu.make_async_copy(
                buf, out_hbm.at[pl.ds(base, _W)], sem
            ).wait()

        gather_start(0, b0, gs0)
        gather_start(1, b1, gs1)

        @pl.loop(0, _NWIN // 2 - 1)
        def _(i):
            w = 2 * i
            gather_wait(b0, gs0)
            wb_start(w, b0, ws0)
            gather_wait(b1, gs1)
            wb_start(w + 1, b1, ws1)
            wb_wait(b0, ws0)
            gather_start(w + 2, b0, gs0)
            wb_wait(b1, ws1)
            gather_start(w + 3, b1, gs1)

        gather_wait(b0, gs0)
        wb_start(_NWIN - 2, b0, ws0)
        gather_wait(b1, gs1)
        wb_start(_NWIN - 1, b1, ws1)
        wb_wait(b0, ws0)
        wb_wait(b1, ws1)

    return k(table_ext, midx)


def kernel(table, indices, lengths):
    idx32 = indices.astype(jnp.int32)
    table_ext = jnp.pad(table, ((0, 8), (0, _DPAD - _DIM)))
    mask, midx = _mask_and_midx(idx32, lengths.reshape(_BATCH, 1))
    emb = _sc_gather(table_ext, midx.reshape(_B))
    emb = emb[:, :_DIM]
    return emb.reshape(_BATCH, _MAXLEN, _DIM), mask


# speedup vs baseline: 1.0008x; 1.0008x over previous
import functools

import jax
import jax.numpy as jnp
from jax import lax
from jax.experimental import pallas as pl
from jax.experimental.pallas import tpu as pltpu
from jax.experimental.pallas import tpu_sc as plsc

_VOCAB = 100000
_DIM = 300
_BATCH = 4096
_MAXLEN = 50
_B = _BATCH * _MAXLEN
_W = 128


def _mask_and_midx(idx32, len2d):
    def body(idx_ref, len_ref, mask_ref, midx_ref):
        pos = lax.broadcasted_iota(jnp.int32, (_BATCH, _MAXLEN), 1)
        valid = pos < len_ref[...]
        mask_ref[...] = valid.astype(jnp.float32)
        midx_ref[...] = jnp.where(valid, idx_ref[...], _VOCAB)

    return pl.pallas_call(
        body,
        out_shape=(
            jax.ShapeDtypeStruct((_BATCH, _MAXLEN), jnp.float32),
            jax.ShapeDtypeStruct((_BATCH, _MAXLEN), jnp.int32),
        ),
    )(idx32, len2d)


_DPAD = 304
_NSC = 32
_BPW = _B // _NSC
_NWIN = _BPW // _W


def _sc_gather(table_ext, midx):
    mesh = plsc.VectorSubcoreMesh(core_axis_name="c", subcore_axis_name="s")

    @functools.partial(
        pl.kernel,
        out_type=jax.ShapeDtypeStruct((_B, _DPAD), jnp.float32),
        mesh=mesh,
        compiler_params=pltpu.CompilerParams(use_tc_tiling_on_sc=False),
    )
    def k(table_hbm, idx_hbm, out_hbm):
        def body(i_vmem, o_vmem):
            pltpu.sync_copy(table_hbm.at[i_vmem.at[0]], o_vmem)

        pltpu.emit_pipeline(
            body,
            grid=(_B // _W,),
            in_specs=[pl.BlockSpec((1, _W), index_map=lambda i: (0, i))],
            out_specs=[pl.BlockSpec((_W, _DPAD), index_map=lambda i: (i, 0))],
            core_axis_name=("c", "s"),
            dimension_semantics=(pltpu.PARALLEL,),
        )(idx_hbm, out_hbm)

    return k(table_ext, midx)


def kernel(table, indices, lengths):
    idx32 = indices.astype(jnp.int32)
    table_ext = jnp.pad(table, ((0, 8), (0, _DPAD - _DIM)))
    mask, midx = _mask_and_midx(idx32, lengths.reshape(_BATCH, 1))
    emb = _sc_gather(table_ext, midx.reshape(1, _B))
    emb = emb[:, :_DIM]
    return emb.reshape(_BATCH, _MAXLEN, _DIM), mask


# speedup vs baseline: 1.0718x; 1.0710x over previous
import functools

import jax
import jax.numpy as jnp
from jax import lax
from jax.experimental import pallas as pl
from jax.experimental.pallas import tpu as pltpu
from jax.experimental.pallas import tpu_sc as plsc

_VOCAB = 100000
_DIM = 300
_BATCH = 4096
_MAXLEN = 50
_B = _BATCH * _MAXLEN
_W = 128


def _mask_and_midx(idx32, len2d):
    def body(idx_ref, len_ref, mask_ref, midx_ref):
        pos = lax.broadcasted_iota(jnp.int32, (_BATCH, _MAXLEN), 1)
        valid = pos < len_ref[...]
        mask_ref[...] = valid.astype(jnp.float32)
        midx_ref[...] = jnp.where(valid, idx_ref[...], _VOCAB)

    return pl.pallas_call(
        body,
        out_shape=(
            jax.ShapeDtypeStruct((_BATCH, _MAXLEN), jnp.float32),
            jax.ShapeDtypeStruct((_BATCH, _MAXLEN), jnp.int32),
        ),
    )(idx32, len2d)


_DPAD = 384
_NSC = 32
_BPW = _B // _NSC
_NWIN = _BPW // _W


def _sc_gather(table_ext, midx):
    mesh = plsc.VectorSubcoreMesh(core_axis_name="c", subcore_axis_name="s")

    @functools.partial(
        pl.kernel,
        out_type=jax.ShapeDtypeStruct((_B, _DPAD), jnp.float32),
        mesh=mesh,
    )
    def k(table_hbm, idx_hbm, out_hbm):
        def body(i_vmem, o_vmem):
            pltpu.sync_copy(table_hbm.at[i_vmem.at[0]], o_vmem)

        pltpu.emit_pipeline(
            body,
            grid=(_B // _W,),
            in_specs=[pl.BlockSpec((1, _W), index_map=lambda i: (0, i))],
            out_specs=[pl.BlockSpec((_W, _DPAD), index_map=lambda i: (i, 0))],
            core_axis_name=("c", "s"),
            dimension_semantics=(pltpu.PARALLEL,),
        )(idx_hbm, out_hbm)

    return k(table_ext, midx)


def kernel(table, indices, lengths):
    idx32 = indices.astype(jnp.int32)
    table_ext = jnp.pad(table, ((0, 8), (0, _DPAD - _DIM)))
    mask, midx = _mask_and_midx(idx32, lengths.reshape(_BATCH, 1))
    emb = _sc_gather(table_ext, midx.reshape(1, _B))
    emb = emb[:, :_DIM]
    return emb.reshape(_BATCH, _MAXLEN, _DIM), mask


# speedup vs baseline: 4.5449x; 4.2406x over previous
import functools

import jax
import jax.numpy as jnp
from jax import lax
from jax.experimental import pallas as pl
from jax.experimental.pallas import tpu as pltpu
from jax.experimental.pallas import tpu_sc as plsc

_VOCAB = 100000
_DIM = 300
_BATCH = 4096
_MAXLEN = 50
_B = _BATCH * _MAXLEN
_W = 128


def _mask_and_midx(idx32, len2d):
    def body(idx_ref, len_ref, mask_ref, midx_ref):
        pos = lax.broadcasted_iota(jnp.int32, (_BATCH, _MAXLEN), 1)
        row = lax.broadcasted_iota(jnp.int32, (_BATCH, _MAXLEN), 0)
        valid = pos < len_ref[...]
        mask_ref[...] = valid.astype(jnp.float32)
        zrow = _VOCAB + ((row * _MAXLEN + pos) & (_NZERO - 1))
        midx_ref[...] = jnp.where(valid, idx_ref[...], zrow)

    return pl.pallas_call(
        body,
        out_shape=(
            jax.ShapeDtypeStruct((_BATCH, _MAXLEN), jnp.float32),
            jax.ShapeDtypeStruct((_BATCH, _MAXLEN), jnp.int32),
        ),
    )(idx32, len2d)


_DPAD = 384
_NZERO = 4096
_NSC = 32
_BPW = _B // _NSC
_NWIN = _BPW // _W


def _sc_gather(table_ext, midx):
    mesh = plsc.VectorSubcoreMesh(core_axis_name="c", subcore_axis_name="s")

    @functools.partial(
        pl.kernel,
        out_type=jax.ShapeDtypeStruct((_B, _DPAD), jnp.float32),
        mesh=mesh,
    )
    def k(table_hbm, idx_hbm, out_hbm):
        def body(i_vmem, o_vmem):
            pltpu.sync_copy(table_hbm.at[i_vmem.at[0]], o_vmem)

        pltpu.emit_pipeline(
            body,
            grid=(_B // _W,),
            in_specs=[pl.BlockSpec((1, _W), index_map=lambda i: (0, i))],
            out_specs=[pl.BlockSpec((_W, _DPAD), index_map=lambda i: (i, 0))],
            core_axis_name=("c", "s"),
            dimension_semantics=(pltpu.PARALLEL,),
        )(idx_hbm, out_hbm)

    return k(table_ext, midx)


def kernel(table, indices, lengths):
    idx32 = indices.astype(jnp.int32)
    table_ext = jnp.pad(table, ((0, _NZERO), (0, _DPAD - _DIM)))
    mask, midx = _mask_and_midx(idx32, lengths.reshape(_BATCH, 1))
    emb = _sc_gather(table_ext, midx.reshape(1, _B))
    emb = emb[:, :_DIM]
    return emb.reshape(_BATCH, _MAXLEN, _DIM), mask


# speedup vs baseline: 5.8998x; 1.2981x over previous
import functools

import jax
import jax.numpy as jnp
from jax import lax
from jax.experimental import pallas as pl
from jax.experimental.pallas import tpu as pltpu
from jax.experimental.pallas import tpu_sc as plsc

_VOCAB = 100000
_DIM = 300
_BATCH = 4096
_MAXLEN = 50
_B = _BATCH * _MAXLEN
_DPAD = 384
_NZERO = 4096
_VEXT = 106000
_PADBLK = 2000
_SLCBLK = 64


def _pad_table(table):
    def body(t_ref, o_ref):
        i = pl.program_id(0)
        row = i * _PADBLK + lax.broadcasted_iota(jnp.int32, (_PADBLK, _DPAD), 0)
        data = jnp.concatenate(
            [t_ref[...], jnp.zeros((_PADBLK, _DPAD - _DIM), jnp.float32)], axis=1
        )
        o_ref[...] = jnp.where(row < _VOCAB, data, 0.0)

    nin = _VOCAB // _PADBLK
    return pl.pallas_call(
        body,
        grid=(_VEXT // _PADBLK,),
        in_specs=[
            pl.BlockSpec(
                (_PADBLK, _DIM), lambda i: (jnp.minimum(i, nin - 1), 0)
            )
        ],
        out_specs=pl.BlockSpec((_PADBLK, _DPAD), lambda i: (i, 0)),
        out_shape=jax.ShapeDtypeStruct((_VEXT, _DPAD), jnp.float32),
    )(table)


def _mask_and_midx(idx32, len2d):
    def body(idx_ref, len_ref, mask_ref, midx_ref):
        pos = lax.broadcasted_iota(jnp.int32, (_BATCH, _MAXLEN), 1)
        row = lax.broadcasted_iota(jnp.int32, (_BATCH, _MAXLEN), 0)
        valid = pos < len_ref[...]
        mask_ref[...] = valid.astype(jnp.float32)
        zrow = _VOCAB + ((row * _MAXLEN + pos) & (_NZERO - 1))
        midx_ref[...] = jnp.where(valid, idx_ref[...], zrow)

    return pl.pallas_call(
        body,
        out_shape=(
            jax.ShapeDtypeStruct((_BATCH, _MAXLEN), jnp.float32),
            jax.ShapeDtypeStruct((_BATCH, _MAXLEN), jnp.int32),
        ),
    )(idx32, len2d)


def _sc_gather(table_ext, midx):
    mesh = plsc.VectorSubcoreMesh(core_axis_name="c", subcore_axis_name="s")

    @functools.partial(
        pl.kernel,
        out_type=jax.ShapeDtypeStruct((_BATCH, _MAXLEN, _DPAD), jnp.float32),
        mesh=mesh,
    )
    def k(table_hbm, idx_hbm, out_hbm):
        def body(i_vmem, o_vmem):
            pltpu.sync_copy(table_hbm.at[i_vmem.at[0]], o_vmem.at[0])

        pltpu.emit_pipeline(
            body,
            grid=(_BATCH,),
            in_specs=[pl.BlockSpec((1, _MAXLEN), index_map=lambda i: (i, 0))],
            out_specs=[
                pl.BlockSpec((1, _MAXLEN, _DPAD), index_map=lambda i: (i, 0, 0))
            ],
            core_axis_name=("c", "s"),
            dimension_semantics=(pltpu.PARALLEL,),
        )(idx_hbm, out_hbm)

    return k(table_ext, midx)


def _slice_out(emb_pad):
    def body(i_ref, o_ref):
        o_ref[...] = i_ref[:, :, : _DIM]

    return pl.pallas_call(
        body,
        grid=(_BATCH // _SLCBLK,),
        in_specs=[
            pl.BlockSpec((_SLCBLK, _MAXLEN, _DPAD), lambda i: (i, 0, 0))
        ],
        out_specs=pl.BlockSpec((_SLCBLK, _MAXLEN, _DIM), lambda i: (i, 0, 0)),
        out_shape=jax.ShapeDtypeStruct((_BATCH, _MAXLEN, _DIM), jnp.float32),
    )(emb_pad)


def kernel(table, indices, lengths):
    idx32 = indices.astype(jnp.int32)
    table_ext = _pad_table(table)
    mask, midx = _mask_and_midx(idx32, lengths.reshape(_BATCH, 1))
    emb_pad = _sc_gather(table_ext, midx)
    emb = _slice_out(emb_pad)
    return emb, mask
